# baseline (speedup 1.0000x reference)
import jax
import jax.numpy as jnp
from jax.experimental import pallas as pl
from jax.experimental.pallas import tpu as pltpu

N = 10000
BM = 200
NI = N // BM

_F32 = jnp.float32


def _dot(a, b):
    return jax.lax.dot_general(a, b, (((1,), (0,)), ((), ())),
                               preferred_element_type=_F32)


def _conv3(x, wa, wb, wc, b):
    z = jnp.zeros((x.shape[0], 1), _F32)
    p = jnp.concatenate([z, x, z], axis=1)
    return (_dot(wa, p[:, 0:64]) + _dot(wb, p[:, 1:65])
            + _dot(wc, p[:, 2:66]) + b)


def _ybranch_body(emb_ref, *refs):
    out_ref = refs[-1]
    w = [r[...] for r in refs[:-1]]
    x = emb_ref[...]
    for j in range(4):
        wa, wb, wc, b1, va, vb, vc, b2, ws, bs = w[10 * j:10 * j + 10]
        h = jnp.maximum(_conv3(x, wa, wb, wc, b1), 0.0)
        h = _conv3(h, va, vb, vc, b2)
        s = _dot(ws, x) + bs
        x = jnp.maximum(h + s, 0.0)
    out_ref[...] = x


def _ybranch(embeddings, rbs):
    args = []
    for (w1, b1, w2, b2, ws, bs) in rbs:
        args += [w1[:, :, 0], w1[:, :, 1], w1[:, :, 2], b1[:, None],
                 w2[:, :, 0], w2[:, :, 1], w2[:, :, 2], b2[:, None],
                 ws[:, :, 0], bs[:, None]]
    return pl.pallas_call(
        _ybranch_body,
        out_shape=jax.ShapeDtypeStruct((9, 64), _F32),
    )(embeddings, *args)


def _g1_body(a_ref, x_ref, w_ref, b_ref, o_ref):
    h = _dot(a_ref[...], x_ref[...])
    o_ref[...] = jnp.maximum(_dot(h, w_ref[...]) + b_ref[...], 0.0)


def _g1(edges, X0, W1, b1):
    return pl.pallas_call(
        _g1_body,
        grid=(NI,),
        in_specs=[pl.BlockSpec((BM, N), lambda i: (i, 0)),
                  pl.BlockSpec((N, 512), lambda i: (0, 0)),
                  pl.BlockSpec((512, 256), lambda i: (0, 0)),
                  pl.BlockSpec((1, 256), lambda i: (0, 0))],
        out_specs=pl.BlockSpec((BM, 256), lambda i: (i, 0)),
        out_shape=jax.ShapeDtypeStruct((N, 256), _F32),
        compiler_params=pltpu.CompilerParams(
            dimension_semantics=("parallel",)),
    )(edges, X0, W1, b1)


def _g2_body(a_ref, x_ref, w_ref, b_ref, o_ref):
    h = _dot(a_ref[...], x_ref[...])
    x2 = jnp.maximum(_dot(h, w_ref[...]) + b_ref[...], 0.0)
    z = jnp.zeros((BM, 1), _F32)
    p = jnp.concatenate([z, x2, z], axis=1)
    m5 = p[:, 0:255]
    for s in range(1, 5):
        m5 = jnp.maximum(m5, p[:, s:s + 255])
    u = jax.lax.broadcasted_iota(jnp.int32, (255, 128), 0)
    t = jax.lax.broadcasted_iota(jnp.int32, (255, 128), 1)
    sel = (u == 2 * t).astype(_F32)
    o_ref[...] = _dot(m5, sel)


def _g2(edges, X1, W2, b2):
    return pl.pallas_call(
        _g2_body,
        grid=(NI,),
        in_specs=[pl.BlockSpec((BM, N), lambda i: (i, 0)),
                  pl.BlockSpec((N, 256), lambda i: (0, 0)),
                  pl.BlockSpec((256, 257), lambda i: (0, 0)),
                  pl.BlockSpec((1, 257), lambda i: (0, 0))],
        out_specs=pl.BlockSpec((BM, 128), lambda i: (i, 0)),
        out_shape=jax.ShapeDtypeStruct((N, 128), _F32),
        compiler_params=pltpu.CompilerParams(
            dimension_semantics=("parallel",)),
    )(edges, X1, W2, b2)


def _g3_body(a_ref, p_ref, w_ref, b_ref, y_ref, fw_ref, fb_ref, lab_ref,
             loss_ref, pred_ref,
             m_ref, s_ref, bs_ref, bi_ref, bl_ref, tgt_ref):
    i = pl.program_id(0)

    @pl.when(i == 0)
    def _():
        m_ref[...] = jnp.full((1, 1), -jnp.inf, _F32)
        s_ref[...] = jnp.zeros((1, 1), _F32)
        bs_ref[...] = jnp.full((1, 1), -jnp.inf, _F32)
        bi_ref[...] = jnp.zeros((1, 1), jnp.int32)
        bl_ref[...] = jnp.full((1, 1), -jnp.inf, _F32)
        tgt_ref[...] = jnp.zeros((1, 1), _F32)

    h = _dot(a_ref[...], p_ref[...])
    x3 = _dot(h, w_ref[...]) + b_ref[...]
    yv = _dot(y_ref[...], fw_ref[...]) + fb_ref[...]
    yr = jnp.broadcast_to(yv[:, 0][None, :], (BM, 64))
    num = jnp.sum(x3 * yr, axis=1, keepdims=True)
    yn = jnp.sqrt(jnp.sum(yr * yr, axis=1, keepdims=True))
    xn = jnp.sqrt(jnp.sum(x3 * x3, axis=1, keepdims=True))
    sim = num / jnp.maximum(xn * yn, 1e-8)

    rows = jax.lax.broadcasted_iota(jnp.int32, (BM, 1), 0)
    lm = jnp.max(sim)
    am = jnp.min(jnp.where(sim == lm, rows, N))
    ls = jnp.sum(jnp.exp(sim - lm))

    mm = m_ref[...]
    new_m = jnp.maximum(mm, lm)
    s_ref[...] = s_ref[...] * jnp.exp(mm - new_m) + ls * jnp.exp(lm - new_m)
    m_ref[...] = new_m

    @pl.when(lm > bs_ref[0, 0])
    def _():
        bs_ref[...] = jnp.full((1, 1), lm, _F32)
        bi_ref[...] = jnp.full((1, 1), i * BM + am, jnp.int32)

    lab = lab_ref[...]
    ll = jnp.max(lab)
    la = jnp.min(jnp.where(lab == ll, rows, N))

    @pl.when(ll > bl_ref[0, 0])
    def _():
        bl_ref[...] = jnp.full((1, 1), ll, _F32)
        tgt_ref[...] = jnp.full(
            (1, 1), jnp.sum(jnp.where(rows == la, sim, 0.0)), _F32)

    @pl.when(i == NI - 1)
    def _():
        loss_ref[...] = (m_ref[...] + jnp.log(s_ref[...])) - tgt_ref[...]
        pred_ref[...] = bi_ref[...]


def _g3(edges, pooled, W3, b3, y649, fc1_w, fc1_b, labels_col):
    return pl.pallas_call(
        _g3_body,
        grid=(NI,),
        in_specs=[pl.BlockSpec((BM, N), lambda i: (i, 0)),
                  pl.BlockSpec((N, 128), lambda i: (0, 0)),
                  pl.BlockSpec((128, 64), lambda i: (0, 0)),
                  pl.BlockSpec((1, 64), lambda i: (0, 0)),
                  pl.BlockSpec((64, 9), lambda i: (0, 0)),
                  pl.BlockSpec((9, 1), lambda i: (0, 0)),
                  pl.BlockSpec((1, 1), lambda i: (0, 0)),
                  pl.BlockSpec((BM, 1), lambda i: (i, 0))],
        out_specs=[pl.BlockSpec((1, 1), lambda i: (0, 0)),
                   pl.BlockSpec((1, 1), lambda i: (0, 0))],
        out_shape=[jax.ShapeDtypeStruct((1, 1), _F32),
                   jax.ShapeDtypeStruct((1, 1), jnp.int32)],
        scratch_shapes=[pltpu.VMEM((1, 1), _F32),
                        pltpu.VMEM((1, 1), _F32),
                        pltpu.VMEM((1, 1), _F32),
                        pltpu.VMEM((1, 1), jnp.int32),
                        pltpu.VMEM((1, 1), _F32),
                        pltpu.VMEM((1, 1), _F32)],
        compiler_params=pltpu.CompilerParams(
            dimension_semantics=("arbitrary",)),
    )(edges, pooled, W3, b3, y649, fc1_w, fc1_b, labels_col)


def kernel(edges, embeddings, labels, node_embed, W1, b1, W2, b2, W3, b3,
           rb1_w1, rb1_b1, rb1_w2, rb1_b2, rb1_ws, rb1_bs,
           rb2_w1, rb2_b1, rb2_w2, rb2_b2, rb2_ws, rb2_bs,
           rb3_w1, rb3_b1, rb3_w2, rb3_b2, rb3_ws, rb3_bs,
           rb4_w1, rb4_b1, rb4_w2, rb4_b2, rb4_ws, rb4_bs,
           fc1_w, fc1_b):
    rbs = [(rb1_w1, rb1_b1, rb1_w2, rb1_b2, rb1_ws, rb1_bs),
           (rb2_w1, rb2_b1, rb2_w2, rb2_b2, rb2_ws, rb2_bs),
           (rb3_w1, rb3_b1, rb3_w2, rb3_b2, rb3_ws, rb3_bs),
           (rb4_w1, rb4_b1, rb4_w2, rb4_b2, rb4_ws, rb4_bs)]
    y9 = _ybranch(embeddings.astype(_F32), rbs)
    y649 = jnp.reshape(y9, (64, 9))
    X1 = _g1(edges, node_embed, W1, b1[None, :])
    pooled = _g2(edges, X1, W2, b2[None, :])
    loss11, preds = _g3(edges, pooled, W3, b3[None, :], y649, fc1_w,
                        jnp.reshape(fc1_b, (1, 1)),
                        jnp.reshape(labels, (N, 1)))
    return jnp.reshape(loss11, ()), preds


# speedup vs baseline: 1.0673x; 1.0673x over previous
import jax
import jax.numpy as jnp
from jax.experimental import pallas as pl
from jax.experimental.pallas import tpu as pltpu

N = 10000
BM = 200
NI = N // BM
BM2 = 400
NI2 = N // BM2

_F32 = jnp.float32


def _dot(a, b):
    return jax.lax.dot_general(a, b, (((1,), (0,)), ((), ())),
                               preferred_element_type=_F32)


def _conv3(x, wa, wb, wc, b):
    z = jnp.zeros((x.shape[0], 1), _F32)
    p = jnp.concatenate([z, x, z], axis=1)
    return (_dot(wa, p[:, 0:64]) + _dot(wb, p[:, 1:65])
            + _dot(wc, p[:, 2:66]) + b)


def _ybranch_body(emb_ref, *refs):
    out_ref = refs[-1]
    w = [r[...] for r in refs[:-1]]
    x = emb_ref[...]
    for j in range(4):
        wa, wb, wc, b1, va, vb, vc, b2, ws, bs = w[10 * j:10 * j + 10]
        h = jnp.maximum(_conv3(x, wa, wb, wc, b1), 0.0)
        h = _conv3(h, va, vb, vc, b2)
        s = _dot(ws, x) + bs
        x = jnp.maximum(h + s, 0.0)
    out_ref[...] = x


def _ybranch(embeddings, rbs):
    args = []
    for (w1, b1, w2, b2, ws, bs) in rbs:
        args += [w1[:, :, 0], w1[:, :, 1], w1[:, :, 2], b1[:, None],
                 w2[:, :, 0], w2[:, :, 1], w2[:, :, 2], b2[:, None],
                 ws[:, :, 0], bs[:, None]]
    return pl.pallas_call(
        _ybranch_body,
        out_shape=jax.ShapeDtypeStruct((9, 64), _F32),
    )(embeddings, *args)


def _g1_body(a_ref, x_ref, w_ref, b_ref, o_ref):
    h = _dot(a_ref[...], x_ref[...])
    o_ref[...] = jnp.maximum(_dot(h, w_ref[...]) + b_ref[...], 0.0)


def _g1(edges, X0, W1, b1):
    return pl.pallas_call(
        _g1_body,
        grid=(NI,),
        in_specs=[pl.BlockSpec((BM, N), lambda i: (i, 0)),
                  pl.BlockSpec((N, 512), lambda i: (0, 0)),
                  pl.BlockSpec((512, 256), lambda i: (0, 0)),
                  pl.BlockSpec((1, 256), lambda i: (0, 0))],
        out_specs=pl.BlockSpec((BM, 256), lambda i: (i, 0)),
        out_shape=jax.ShapeDtypeStruct((N, 256), _F32),
        compiler_params=pltpu.CompilerParams(
            dimension_semantics=("parallel",)),
    )(edges, X0, W1, b1)


def _g2_body(a_ref, x_ref, w_ref, b_ref, o_ref):
    h = _dot(a_ref[...], x_ref[...])
    x2 = jnp.maximum(_dot(h, w_ref[...]) + b_ref[...], 0.0)
    z = jnp.zeros((BM2, 1), _F32)
    p = jnp.concatenate([z, x2, z], axis=1)
    m5 = p[:, 0:255]
    for s in range(1, 5):
        m5 = jnp.maximum(m5, p[:, s:s + 255])
    u = jax.lax.broadcasted_iota(jnp.int32, (255, 128), 0)
    t = jax.lax.broadcasted_iota(jnp.int32, (255, 128), 1)
    sel = (u == 2 * t).astype(_F32)
    o_ref[...] = _dot(m5, sel)


def _g2(edges, X1, W2, b2):
    return pl.pallas_call(
        _g2_body,
        grid=(NI2,),
        in_specs=[pl.BlockSpec((BM2, N), lambda i: (i, 0)),
                  pl.BlockSpec((N, 256), lambda i: (0, 0)),
                  pl.BlockSpec((256, 257), lambda i: (0, 0)),
                  pl.BlockSpec((1, 257), lambda i: (0, 0))],
        out_specs=pl.BlockSpec((BM2, 128), lambda i: (i, 0)),
        out_shape=jax.ShapeDtypeStruct((N, 128), _F32),
        compiler_params=pltpu.CompilerParams(
            dimension_semantics=("parallel",)),
    )(edges, X1, W2, b2)


def _g3_body(a_ref, p_ref, w_ref, b_ref, y_ref, fw_ref, fb_ref, lab_ref,
             loss_ref, pred_ref,
             m_ref, s_ref, bs_ref, bi_ref, bl_ref, tgt_ref):
    i = pl.program_id(0)

    @pl.when(i == 0)
    def _():
        m_ref[...] = jnp.full((1, 1), -jnp.inf, _F32)
        s_ref[...] = jnp.zeros((1, 1), _F32)
        bs_ref[...] = jnp.full((1, 1), -jnp.inf, _F32)
        bi_ref[...] = jnp.zeros((1, 1), jnp.int32)
        bl_ref[...] = jnp.full((1, 1), -jnp.inf, _F32)
        tgt_ref[...] = jnp.zeros((1, 1), _F32)

    h = _dot(a_ref[...], p_ref[...])
    x3 = _dot(h, w_ref[...]) + b_ref[...]
    yv = _dot(y_ref[...], fw_ref[...]) + fb_ref[...]
    yr = jnp.broadcast_to(yv[:, 0][None, :], (BM2, 64))
    num = jnp.sum(x3 * yr, axis=1, keepdims=True)
    yn = jnp.sqrt(jnp.sum(yr * yr, axis=1, keepdims=True))
    xn = jnp.sqrt(jnp.sum(x3 * x3, axis=1, keepdims=True))
    sim = num / jnp.maximum(xn * yn, 1e-8)

    rows = jax.lax.broadcasted_iota(jnp.int32, (BM2, 1), 0)
    lm = jnp.max(sim)
    am = jnp.min(jnp.where(sim == lm, rows, N))
    ls = jnp.sum(jnp.exp(sim - lm))

    mm = m_ref[...]
    new_m = jnp.maximum(mm, lm)
    s_ref[...] = s_ref[...] * jnp.exp(mm - new_m) + ls * jnp.exp(lm - new_m)
    m_ref[...] = new_m

    @pl.when(lm > bs_ref[0, 0])
    def _():
        bs_ref[...] = jnp.full((1, 1), lm, _F32)
        bi_ref[...] = jnp.full((1, 1), i * BM2 + am, jnp.int32)

    lab = lab_ref[...]
    ll = jnp.max(lab)
    la = jnp.min(jnp.where(lab == ll, rows, N))

    @pl.when(ll > bl_ref[0, 0])
    def _():
        bl_ref[...] = jnp.full((1, 1), ll, _F32)
        tgt_ref[...] = jnp.full(
            (1, 1), jnp.sum(jnp.where(rows == la, sim, 0.0)), _F32)

    @pl.when(i == NI2 - 1)
    def _():
        loss_ref[...] = (m_ref[...] + jnp.log(s_ref[...])) - tgt_ref[...]
        pred_ref[...] = bi_ref[...]


def _g3(edges, pooled, W3, b3, y649, fc1_w, fc1_b, labels_col):
    return pl.pallas_call(
        _g3_body,
        grid=(NI2,),
        in_specs=[pl.BlockSpec((BM2, N), lambda i: (i, 0)),
                  pl.BlockSpec((N, 128), lambda i: (0, 0)),
                  pl.BlockSpec((128, 64), lambda i: (0, 0)),
                  pl.BlockSpec((1, 64), lambda i: (0, 0)),
                  pl.BlockSpec((64, 9), lambda i: (0, 0)),
                  pl.BlockSpec((9, 1), lambda i: (0, 0)),
                  pl.BlockSpec((1, 1), lambda i: (0, 0)),
                  pl.BlockSpec((BM2, 1), lambda i: (i, 0))],
        out_specs=[pl.BlockSpec((1, 1), lambda i: (0, 0)),
                   pl.BlockSpec((1, 1), lambda i: (0, 0))],
        out_shape=[jax.ShapeDtypeStruct((1, 1), _F32),
                   jax.ShapeDtypeStruct((1, 1), jnp.int32)],
        scratch_shapes=[pltpu.VMEM((1, 1), _F32),
                        pltpu.VMEM((1, 1), _F32),
                        pltpu.VMEM((1, 1), _F32),
                        pltpu.VMEM((1, 1), jnp.int32),
                        pltpu.VMEM((1, 1), _F32),
                        pltpu.VMEM((1, 1), _F32)],
        compiler_params=pltpu.CompilerParams(
            dimension_semantics=("arbitrary",)),
    )(edges, pooled, W3, b3, y649, fc1_w, fc1_b, labels_col)


def kernel(edges, embeddings, labels, node_embed, W1, b1, W2, b2, W3, b3,
           rb1_w1, rb1_b1, rb1_w2, rb1_b2, rb1_ws, rb1_bs,
           rb2_w1, rb2_b1, rb2_w2, rb2_b2, rb2_ws, rb2_bs,
           rb3_w1, rb3_b1, rb3_w2, rb3_b2, rb3_ws, rb3_bs,
           rb4_w1, rb4_b1, rb4_w2, rb4_b2, rb4_ws, rb4_bs,
           fc1_w, fc1_b):
    rbs = [(rb1_w1, rb1_b1, rb1_w2, rb1_b2, rb1_ws, rb1_bs),
           (rb2_w1, rb2_b1, rb2_w2, rb2_b2, rb2_ws, rb2_bs),
           (rb3_w1, rb3_b1, rb3_w2, rb3_b2, rb3_ws, rb3_bs),
           (rb4_w1, rb4_b1, rb4_w2, rb4_b2, rb4_ws, rb4_bs)]
    y9 = _ybranch(embeddings.astype(_F32), rbs)
    y649 = jnp.reshape(y9, (64, 9))
    X1 = _g1(edges, node_embed, W1, b1[None, :])
    pooled = _g2(edges, X1, W2, b2[None, :])
    loss11, preds = _g3(edges, pooled, W3, b3[None, :], y649, fc1_w,
                        jnp.reshape(fc1_b, (1, 1)),
                        jnp.reshape(labels, (N, 1)))
    return jnp.reshape(loss11, ()), preds
